# baseline (speedup 1.0000x reference)
import functools

import jax
from jax import lax
import jax.numpy as jnp
from jax.experimental import pallas as pl
import jax.experimental.pallas.tpu as pltpu
from jax.experimental.pallas import tpu_sc as plsc

N = 262144
B = 8
NUM_LM = 256
C = 16384
NCH = N // C
V = NCH + B - 1


def _topk_kernel(sref, lm_ref, pts_ref, bidx_ref, ovals_ref, oidx_ref):
    v = pl.program_id(0)
    vb = sref[v]
    vc = sref[V + v]

    prev_vb = sref[jnp.maximum(v - 1, 0)]
    first = jnp.logical_or(v == 0, vb != prev_vb)

    @pl.when(jnp.logical_and(first, vb < B))
    def _init():
        ovals_ref[...] = jnp.full((NUM_LM, 3), jnp.inf, dtype=jnp.float32)
        oidx_ref[...] = jnp.zeros((NUM_LM, 3), dtype=jnp.int32)

    @pl.when(vb < B)
    def _compute():
        px = pts_ref[0:1, :]
        py = pts_ref[1:2, :]
        pz = pts_ref[2:3, :]
        lx = lm_ref[:, 0:1]
        ly = lm_ref[:, 1:2]
        lz = lm_ref[:, 2:3]
        dx = lx - px
        dy = ly - py
        dz = lz - pz
        d = dx * dx + dy * dy + dz * dz
        in_seg = bidx_ref[0:1, :] == vb
        d = jnp.where(in_seg, d, jnp.inf)
        cidx = (jax.lax.broadcasted_iota(jnp.int32, (1, C), 1)
                + vc * C).astype(jnp.float32)
        BIGF = jnp.float32(1e9)

        r1v = ovals_ref[:, 0:1]
        r2v = ovals_ref[:, 1:2]
        r3v = ovals_ref[:, 2:3]
        r1i = oidx_ref[:, 0:1]
        r2i = oidx_ref[:, 1:2]
        r3i = oidx_ref[:, 2:3]

        for t in range(3):
            m = jnp.min(d, axis=1, keepdims=True)
            imf = jnp.min(jnp.where(d == m, cidx, BIGF),
                          axis=1, keepdims=True)
            if t < 2:
                d = jnp.where(cidx == imf, jnp.inf, d)
            im = imf.astype(jnp.int32)
            c3 = m < r3v
            v3 = jnp.where(c3, m, r3v)
            i3 = jnp.where(c3, im, r3i)
            c2 = v3 < r2v
            nv2 = jnp.where(c2, v3, r2v)
            ni2 = jnp.where(c2, i3, r2i)
            r3v = jnp.where(c2, r2v, v3)
            r3i = jnp.where(c2, r2i, i3)
            c1 = nv2 < r1v
            nv1 = jnp.where(c1, nv2, r1v)
            ni1 = jnp.where(c1, ni2, r1i)
            r2v = jnp.where(c1, r1v, nv2)
            r2i = jnp.where(c1, r1i, ni2)
            r1v = nv1
            r1i = ni1

        ovals_ref[:, 0:1] = r1v
        ovals_ref[:, 1:2] = r2v
        ovals_ref[:, 2:3] = r3v
        oidx_ref[:, 0:1] = r1i
        oidx_ref[:, 1:2] = r2i
        oidx_ref[:, 2:3] = r3i


_NC = 2
_NS = 16
_NW = _NC * _NS
_LPW = (B * NUM_LM) // _NW
_EPW = _LPW * 3
_GPW = _LPW // 16


def _sc_combine(idx_hbm, vals_hbm, pdis_hbm, plm_hbm, tlm_hbm,
                out_hbm, loss_hbm,
                idx_v, row_v, vals_v, plm_v, tlm_v, rows_v, out_v, acc_v, sem):
    wid = lax.axis_index("s") * _NC + lax.axis_index("c")
    base = wid * _EPW

    pltpu.sync_copy(plm_hbm.at[pl.ds(base, _EPW)], plm_v)
    pltpu.sync_copy(tlm_hbm.at[pl.ds(base, _EPW)], tlm_v)
    pltpu.sync_copy(vals_hbm.at[pl.ds(base, _EPW)], vals_v)
    pltpu.sync_copy(idx_hbm.at[pl.ds(base, _EPW)], idx_v)

    for i in range(_EPW // 16):
        row_v[i // 6, pl.ds((i % 6) * 16, 16)] = idx_v[pl.ds(i * 16, 16)] >> 5

    pltpu.async_copy(pdis_hbm.at[row_v.at[0]], rows_v.at[pl.ds(0, 96)], sem)
    pltpu.async_copy(pdis_hbm.at[row_v.at[1]], rows_v.at[pl.ds(96, 96)], sem)
    pltpu.make_async_copy(pdis_hbm.at[row_v.at[0]],
                          rows_v.at[pl.ds(0, 96)], sem).wait()
    pltpu.make_async_copy(pdis_hbm.at[row_v.at[1]],
                          rows_v.at[pl.ds(96, 96)], sem).wait()

    acc = jnp.zeros((16,), jnp.float32)
    lane = lax.iota(jnp.int32, 16)
    for g in range(_GPW):
        e0 = g * 48 + 3 * lane
        v = [plsc.load_gather(vals_v, [e0 + t]) for t in range(3)]
        k = [1.0 / (vt + 1e-8) for vt in v]
        s = k[0] + k[1] + k[2]
        w = [kt / s for kt in k]
        col = [(plsc.load_gather(idx_v, [e0 + t]) & 31) << 2 for t in range(3)]
        for c in range(3):
            p_c = (w[0] * plsc.load_gather(rows_v, [e0 + 0, col[0] + c])
                   + w[1] * plsc.load_gather(rows_v, [e0 + 1, col[1] + c])
                   + w[2] * plsc.load_gather(rows_v, [e0 + 2, col[2] + c]))
            pred_c = plsc.load_gather(plm_v, [e0 + c]) + p_c
            plsc.store_scatter(out_v, [e0 + c], pred_c)
            err = plsc.load_gather(tlm_v, [e0 + c]) - pred_c
            acc = acc + err * err

    acc_v[...] = acc
    pltpu.sync_copy(out_v, out_hbm.at[pl.ds(base, _EPW)])
    pltpu.sync_copy(acc_v, loss_hbm.at[pl.ds(wid * 16, 16)])


_PACK_BLK = 512


def _sc_pack(pdis_hbm, out_hbm, in_v, out_v):
    wid = lax.axis_index("s") * _NC + lax.axis_index("c")
    ppw = N // _NW
    lane = lax.iota(jnp.int32, 16)
    lp = lane >> 2
    lc = jnp.minimum(lane & 3, 2)
    for blk in range(ppw // _PACK_BLK):
        src = wid * ppw + blk * _PACK_BLK
        dst = wid * (ppw // 32) + blk * (_PACK_BLK // 32)
        pltpu.sync_copy(pdis_hbm.at[pl.ds(src, _PACK_BLK), :], in_v)

        def body(r, _):
            p0 = r * 32
            for k in range(8):
                out_v[r, pl.ds(k * 16, 16)] = plsc.load_gather(
                    in_v, [p0 + 4 * k + lp, lc])
            return 0

        lax.fori_loop(0, _PACK_BLK // 32, body, 0)
        pltpu.sync_copy(
            out_v, out_hbm.at[pl.ds(dst, _PACK_BLK // 32), :])


_SC_CALL = None
_SC_PACK_CALL = None


def _get_sc_pack():
    global _SC_PACK_CALL
    if _SC_PACK_CALL is None:
        _SC_PACK_CALL = functools.partial(
            pl.kernel,
            mesh=plsc.VectorSubcoreMesh(core_axis_name="c",
                                        subcore_axis_name="s",
                                        num_cores=_NC, num_subcores=_NS),
            out_type=jax.ShapeDtypeStruct((N // 32, 128), jnp.float32),
            scratch_types=[
                pltpu.VMEM((_PACK_BLK, 3), jnp.float32),
                pltpu.VMEM((_PACK_BLK // 32, 128), jnp.float32),
            ],
            compiler_params=pltpu.CompilerParams(needs_layout_passes=False),
        )(_sc_pack)
    return _SC_PACK_CALL


def _get_sc_combine():
    global _SC_CALL
    if _SC_CALL is None:
        _SC_CALL = functools.partial(
            pl.kernel,
            mesh=plsc.VectorSubcoreMesh(core_axis_name="c",
                                        subcore_axis_name="s",
                                        num_cores=_NC, num_subcores=_NS),
            out_type=[
                jax.ShapeDtypeStruct((B * NUM_LM * 3,), jnp.float32),
                jax.ShapeDtypeStruct((_NW * 16,), jnp.float32),
            ],
            scratch_types=[
                pltpu.VMEM((_EPW,), jnp.int32),
                pltpu.VMEM((2, 96), jnp.int32),
                pltpu.VMEM((_EPW,), jnp.float32),
                pltpu.VMEM((_EPW,), jnp.float32),
                pltpu.VMEM((_EPW,), jnp.float32),
                pltpu.VMEM((_EPW, 128), jnp.float32),
                pltpu.VMEM((_EPW,), jnp.float32),
                pltpu.VMEM((16,), jnp.float32),
                pltpu.SemaphoreType.DMA,
            ],
            compiler_params=pltpu.CompilerParams(needs_layout_passes=False),
        )(_sc_combine)
    return _SC_CALL


def _loss_reduce_kernel(parts_ref, loss_ref):
    loss_ref[0, 0] = jnp.sum(parts_ref[...]) / (B * NUM_LM)


def kernel(pred_dis, pre_xyz, pre_lm, target_lm, batch):
    batch = batch.astype(jnp.int32)
    pts_t = pre_xyz.T
    batch2d = batch.reshape(1, N)

    bids = jnp.arange(B, dtype=jnp.int32)
    starts = jnp.searchsorted(batch, bids, side="left").astype(jnp.int32)
    ends = jnp.searchsorted(batch, bids, side="right").astype(jnp.int32)
    start_c = starts // C
    end_c = jnp.where(ends > starts, (ends + C - 1) // C, start_c)
    counts = end_c - start_c
    cum = jnp.cumsum(counts)
    off = jnp.concatenate([jnp.zeros((1,), jnp.int32), cum[:-1]])
    vs = jnp.arange(V, dtype=jnp.int32)
    vb = jnp.searchsorted(cum, vs, side="right").astype(jnp.int32)
    vbc = jnp.minimum(vb, B - 1)
    vc = jnp.clip(start_c[vbc] + vs - off[vbc], 0, NCH - 1)
    scalars = jnp.concatenate([vb, vc])

    grid_spec = pltpu.PrefetchScalarGridSpec(
        num_scalar_prefetch=1,
        grid=(V,),
        in_specs=[
            pl.BlockSpec((NUM_LM, 3), lambda v, s: (jnp.minimum(s[v], B - 1), 0)),
            pl.BlockSpec((3, C), lambda v, s: (0, s[V + v])),
            pl.BlockSpec((1, C), lambda v, s: (0, s[V + v])),
        ],
        out_specs=[
            pl.BlockSpec((NUM_LM, 3), lambda v, s: (jnp.minimum(s[v], B - 1), 0)),
            pl.BlockSpec((NUM_LM, 3), lambda v, s: (jnp.minimum(s[v], B - 1), 0)),
        ],
    )
    top_vals, top_idx = pl.pallas_call(
        _topk_kernel,
        grid_spec=grid_spec,
        out_shape=[
            jax.ShapeDtypeStruct((B * NUM_LM, 3), jnp.float32),
            jax.ShapeDtypeStruct((B * NUM_LM, 3), jnp.int32),
        ],
    )(scalars, pre_lm, pts_t, batch2d)

    pdis_pack = _get_sc_pack()(pred_dis)
    pred_flat, loss_parts = _get_sc_combine()(
        top_idx.reshape(-1),
        top_vals.reshape(-1),
        pdis_pack,
        pre_lm.reshape(-1),
        target_lm.reshape(-1),
    )

    loss = pl.pallas_call(
        _loss_reduce_kernel,
        in_specs=[pl.BlockSpec(memory_space=pltpu.VMEM)],
        out_specs=pl.BlockSpec(memory_space=pltpu.SMEM),
        out_shape=jax.ShapeDtypeStruct((1, 1), jnp.float32),
    )(loss_parts.reshape(_NW, 16))

    return loss[0, 0], pred_flat.reshape(B * NUM_LM, 3)


# speedup vs baseline: 1.1469x; 1.1469x over previous
import functools

import jax
from jax import lax
import jax.numpy as jnp
from jax.experimental import pallas as pl
import jax.experimental.pallas.tpu as pltpu
from jax.experimental.pallas import tpu_sc as plsc

N = 262144
B = 8
NUM_LM = 256
C = 8192
NCH = N // C
V = NCH + B - 1


def _topk_kernel(sref, lm_ref, pts_ref, bidx_ref, ovals_ref, oidx_ref):
    v = pl.program_id(0)
    vb = sref[v]
    vc = sref[V + v]

    prev_vb = sref[jnp.maximum(v - 1, 0)]
    first = jnp.logical_or(v == 0, vb != prev_vb)

    @pl.when(jnp.logical_and(first, vb < B))
    def _init():
        ovals_ref[...] = jnp.full((NUM_LM, 3), jnp.inf, dtype=jnp.float32)
        oidx_ref[...] = jnp.zeros((NUM_LM, 3), dtype=jnp.int32)

    @pl.when(vb < B)
    def _compute():
        px = pts_ref[0:1, :]
        py = pts_ref[1:2, :]
        pz = pts_ref[2:3, :]
        lx = lm_ref[:, 0:1]
        ly = lm_ref[:, 1:2]
        lz = lm_ref[:, 2:3]
        dx = lx - px
        dy = ly - py
        dz = lz - pz
        d = dx * dx + dy * dy + dz * dz
        in_seg = bidx_ref[0:1, :] == vb
        d = jnp.where(in_seg, d, jnp.inf)
        cidx = (jax.lax.broadcasted_iota(jnp.int32, (1, C), 1)
                + vc * C).astype(jnp.float32)
        BIGF = jnp.float32(1e9)

        r1v = ovals_ref[:, 0:1]
        r2v = ovals_ref[:, 1:2]
        r3v = ovals_ref[:, 2:3]
        r1i = oidx_ref[:, 0:1]
        r2i = oidx_ref[:, 1:2]
        r3i = oidx_ref[:, 2:3]

        for t in range(3):
            m = jnp.min(d, axis=1, keepdims=True)
            imf = jnp.min(jnp.where(d == m, cidx, BIGF),
                          axis=1, keepdims=True)
            if t < 2:
                d = jnp.where(cidx == imf, jnp.inf, d)
            im = imf.astype(jnp.int32)
            c3 = m < r3v
            v3 = jnp.where(c3, m, r3v)
            i3 = jnp.where(c3, im, r3i)
            c2 = v3 < r2v
            nv2 = jnp.where(c2, v3, r2v)
            ni2 = jnp.where(c2, i3, r2i)
            r3v = jnp.where(c2, r2v, v3)
            r3i = jnp.where(c2, r2i, i3)
            c1 = nv2 < r1v
            nv1 = jnp.where(c1, nv2, r1v)
            ni1 = jnp.where(c1, ni2, r1i)
            r2v = jnp.where(c1, r1v, nv2)
            r2i = jnp.where(c1, r1i, ni2)
            r1v = nv1
            r1i = ni1

        ovals_ref[:, 0:1] = r1v
        ovals_ref[:, 1:2] = r2v
        ovals_ref[:, 2:3] = r3v
        oidx_ref[:, 0:1] = r1i
        oidx_ref[:, 1:2] = r2i
        oidx_ref[:, 2:3] = r3i


_NC = 2
_NS = 16
_NW = _NC * _NS
_LPW = (B * NUM_LM) // _NW
_EPW = _LPW * 3
_GPW = _LPW // 16


def _sc_combine(idx_hbm, vals_hbm, pdis_hbm, plm_hbm, tlm_hbm,
                out_hbm, loss_hbm,
                idx_v, row_v, vals_v, plm_v, tlm_v, rows_v, out_v, acc_v, sem):
    wid = lax.axis_index("s") * _NC + lax.axis_index("c")
    lbase = wid * _LPW

    pltpu.sync_copy(plm_hbm.at[pl.ds(lbase, _LPW), :], plm_v)
    pltpu.sync_copy(tlm_hbm.at[pl.ds(lbase, _LPW), :], tlm_v)
    pltpu.sync_copy(vals_hbm.at[pl.ds(lbase, _LPW), :], vals_v)
    pltpu.sync_copy(idx_hbm.at[pl.ds(lbase, _LPW), :], idx_v)

    lane = lax.iota(jnp.int32, 16)
    for i in range(_EPW // 16):
        pos = i * 16 + lane
        row_v[i // 6, pl.ds((i % 6) * 16, 16)] = (
            plsc.load_gather(idx_v, [pos // 3, pos % 3]) >> 5)

    pltpu.async_copy(pdis_hbm.at[row_v.at[0]], rows_v.at[pl.ds(0, 96)], sem)
    pltpu.async_copy(pdis_hbm.at[row_v.at[1]], rows_v.at[pl.ds(96, 96)], sem)
    pltpu.make_async_copy(pdis_hbm.at[row_v.at[0]],
                          rows_v.at[pl.ds(0, 96)], sem).wait()
    pltpu.make_async_copy(pdis_hbm.at[row_v.at[1]],
                          rows_v.at[pl.ds(96, 96)], sem).wait()

    acc = jnp.zeros((16,), jnp.float32)
    ccs = [jnp.full((16,), c, jnp.int32) for c in range(3)]
    for g in range(_GPW):
        lidx = g * 16 + lane
        e0 = 3 * lidx
        v = [plsc.load_gather(vals_v, [lidx, ccs[t]]) for t in range(3)]
        k = [1.0 / (vt + 1e-8) for vt in v]
        s = k[0] + k[1] + k[2]
        w = [kt / s for kt in k]
        col = [(plsc.load_gather(idx_v, [lidx, ccs[t]]) & 31) << 2
               for t in range(3)]
        for c in range(3):
            p_c = (w[0] * plsc.load_gather(rows_v, [e0 + 0, col[0] + c])
                   + w[1] * plsc.load_gather(rows_v, [e0 + 1, col[1] + c])
                   + w[2] * plsc.load_gather(rows_v, [e0 + 2, col[2] + c]))
            pred_c = plsc.load_gather(plm_v, [lidx, ccs[c]]) + p_c
            plsc.store_scatter(out_v, [lidx, ccs[c]], pred_c)
            err = plsc.load_gather(tlm_v, [lidx, ccs[c]]) - pred_c
            acc = acc + err * err

    acc_v[...] = acc
    pltpu.sync_copy(out_v, out_hbm.at[pl.ds(lbase, _LPW), :])
    pltpu.sync_copy(acc_v, loss_hbm.at[pl.ds(wid * 16, 16)])


_PACK_BLK = 512


def _sc_pack(pdis_hbm, out_hbm, in_v, out_v):
    wid = lax.axis_index("s") * _NC + lax.axis_index("c")
    ppw = N // _NW
    lane = lax.iota(jnp.int32, 16)
    lp = lane >> 2
    lc = jnp.minimum(lane & 3, 2)
    for blk in range(ppw // _PACK_BLK):
        src = wid * ppw + blk * _PACK_BLK
        dst = wid * (ppw // 32) + blk * (_PACK_BLK // 32)
        pltpu.sync_copy(pdis_hbm.at[pl.ds(src, _PACK_BLK), :], in_v)

        def body(r, _):
            p0 = r * 32
            for k in range(8):
                out_v[r, pl.ds(k * 16, 16)] = plsc.load_gather(
                    in_v, [p0 + 4 * k + lp, lc])
            return 0

        lax.fori_loop(0, _PACK_BLK // 32, body, 0)
        pltpu.sync_copy(
            out_v, out_hbm.at[pl.ds(dst, _PACK_BLK // 32), :])


_SC_CALL = None
_SC_PACK_CALL = None


def _get_sc_pack():
    global _SC_PACK_CALL
    if _SC_PACK_CALL is None:
        _SC_PACK_CALL = functools.partial(
            pl.kernel,
            mesh=plsc.VectorSubcoreMesh(core_axis_name="c",
                                        subcore_axis_name="s",
                                        num_cores=_NC, num_subcores=_NS),
            out_type=jax.ShapeDtypeStruct((N // 32, 128), jnp.float32),
            scratch_types=[
                pltpu.VMEM((_PACK_BLK, 3), jnp.float32),
                pltpu.VMEM((_PACK_BLK // 32, 128), jnp.float32),
            ],
            compiler_params=pltpu.CompilerParams(needs_layout_passes=False),
        )(_sc_pack)
    return _SC_PACK_CALL


def _get_sc_combine():
    global _SC_CALL
    if _SC_CALL is None:
        _SC_CALL = functools.partial(
            pl.kernel,
            mesh=plsc.VectorSubcoreMesh(core_axis_name="c",
                                        subcore_axis_name="s",
                                        num_cores=_NC, num_subcores=_NS),
            out_type=[
                jax.ShapeDtypeStruct((B * NUM_LM, 3), jnp.float32),
                jax.ShapeDtypeStruct((_NW * 16,), jnp.float32),
            ],
            scratch_types=[
                pltpu.VMEM((_LPW, 3), jnp.int32),
                pltpu.VMEM((2, 96), jnp.int32),
                pltpu.VMEM((_LPW, 3), jnp.float32),
                pltpu.VMEM((_LPW, 3), jnp.float32),
                pltpu.VMEM((_LPW, 3), jnp.float32),
                pltpu.VMEM((_EPW, 128), jnp.float32),
                pltpu.VMEM((_LPW, 3), jnp.float32),
                pltpu.VMEM((16,), jnp.float32),
                pltpu.SemaphoreType.DMA,
            ],
            compiler_params=pltpu.CompilerParams(needs_layout_passes=False),
        )(_sc_combine)
    return _SC_CALL


def _loss_reduce_kernel(parts_ref, loss_ref):
    loss_ref[0, 0] = jnp.sum(parts_ref[...]) / (B * NUM_LM)


def kernel(pred_dis, pre_xyz, pre_lm, target_lm, batch):
    batch = batch.astype(jnp.int32)
    pts_t = pre_xyz.T
    batch2d = batch.reshape(1, N)

    bids = jnp.arange(B, dtype=jnp.int32)
    starts = jnp.searchsorted(batch, bids, side="left").astype(jnp.int32)
    ends = jnp.searchsorted(batch, bids, side="right").astype(jnp.int32)
    start_c = starts // C
    end_c = jnp.where(ends > starts, (ends + C - 1) // C, start_c)
    counts = end_c - start_c
    cum = jnp.cumsum(counts)
    off = jnp.concatenate([jnp.zeros((1,), jnp.int32), cum[:-1]])
    vs = jnp.arange(V, dtype=jnp.int32)
    vb = jnp.searchsorted(cum, vs, side="right").astype(jnp.int32)
    vbc = jnp.minimum(vb, B - 1)
    vc = jnp.clip(start_c[vbc] + vs - off[vbc], 0, NCH - 1)
    scalars = jnp.concatenate([vb, vc])

    grid_spec = pltpu.PrefetchScalarGridSpec(
        num_scalar_prefetch=1,
        grid=(V,),
        in_specs=[
            pl.BlockSpec((NUM_LM, 3), lambda v, s: (jnp.minimum(s[v], B - 1), 0)),
            pl.BlockSpec((3, C), lambda v, s: (0, s[V + v])),
            pl.BlockSpec((1, C), lambda v, s: (0, s[V + v])),
        ],
        out_specs=[
            pl.BlockSpec((NUM_LM, 3), lambda v, s: (jnp.minimum(s[v], B - 1), 0)),
            pl.BlockSpec((NUM_LM, 3), lambda v, s: (jnp.minimum(s[v], B - 1), 0)),
        ],
    )
    top_vals, top_idx = pl.pallas_call(
        _topk_kernel,
        grid_spec=grid_spec,
        out_shape=[
            jax.ShapeDtypeStruct((B * NUM_LM, 3), jnp.float32),
            jax.ShapeDtypeStruct((B * NUM_LM, 3), jnp.int32),
        ],
    )(scalars, pre_lm, pts_t, batch2d)

    pdis_pack = _get_sc_pack()(pred_dis)
    pred_lm, loss_parts = _get_sc_combine()(
        top_idx,
        top_vals,
        pdis_pack,
        pre_lm,
        target_lm,
    )

    loss = pl.pallas_call(
        _loss_reduce_kernel,
        in_specs=[pl.BlockSpec(memory_space=pltpu.VMEM)],
        out_specs=pl.BlockSpec(memory_space=pltpu.SMEM),
        out_shape=jax.ShapeDtypeStruct((1, 1), jnp.float32),
    )(loss_parts.reshape(_NW, 16))

    return loss[0, 0], pred_lm


# speedup vs baseline: 1.1480x; 1.0010x over previous
import functools

import jax
from jax import lax
import jax.numpy as jnp
from jax.experimental import pallas as pl
import jax.experimental.pallas.tpu as pltpu
from jax.experimental.pallas import tpu_sc as plsc

N = 262144
B = 8
NUM_LM = 256
C = 8192
NCH = N // C
V = NCH + B - 1


def _topk_kernel(sref, lm_ref, pts_ref, ovals_ref, oidx_ref):
    v = pl.program_id(0)
    vb = sref[v]
    vc = sref[V + v]
    vbc = jnp.minimum(vb, B - 1)
    seg_lo = sref[2 * V + vbc]
    seg_hi = sref[2 * V + B + vbc]

    prev_vb = sref[jnp.maximum(v - 1, 0)]
    first = jnp.logical_or(v == 0, vb != prev_vb)

    @pl.when(jnp.logical_and(first, vb < B))
    def _init():
        ovals_ref[...] = jnp.full((NUM_LM, 3), jnp.inf, dtype=jnp.float32)
        oidx_ref[...] = jnp.zeros((NUM_LM, 3), dtype=jnp.int32)

    @pl.when(vb < B)
    def _compute():
        px = pts_ref[0:1, :]
        py = pts_ref[1:2, :]
        pz = pts_ref[2:3, :]
        lx = lm_ref[:, 0:1]
        ly = lm_ref[:, 1:2]
        lz = lm_ref[:, 2:3]
        dx = lx - px
        dy = ly - py
        dz = lz - pz
        d = dx * dx + dy * dy + dz * dz
        gidx = jax.lax.broadcasted_iota(jnp.int32, (1, C), 1) + vc * C
        in_seg = jnp.logical_and(gidx >= seg_lo, gidx < seg_hi)
        d = jnp.where(in_seg, d, jnp.inf)
        cidx = gidx.astype(jnp.float32)
        BIGF = jnp.float32(1e9)

        r1v = ovals_ref[:, 0:1]
        r2v = ovals_ref[:, 1:2]
        r3v = ovals_ref[:, 2:3]
        r1i = oidx_ref[:, 0:1]
        r2i = oidx_ref[:, 1:2]
        r3i = oidx_ref[:, 2:3]

        for t in range(3):
            m = jnp.min(d, axis=1, keepdims=True)
            imf = jnp.min(jnp.where(d == m, cidx, BIGF),
                          axis=1, keepdims=True)
            if t < 2:
                d = jnp.where(cidx == imf, jnp.inf, d)
            im = imf.astype(jnp.int32)
            c3 = m < r3v
            v3 = jnp.where(c3, m, r3v)
            i3 = jnp.where(c3, im, r3i)
            c2 = v3 < r2v
            nv2 = jnp.where(c2, v3, r2v)
            ni2 = jnp.where(c2, i3, r2i)
            r3v = jnp.where(c2, r2v, v3)
            r3i = jnp.where(c2, r2i, i3)
            c1 = nv2 < r1v
            nv1 = jnp.where(c1, nv2, r1v)
            ni1 = jnp.where(c1, ni2, r1i)
            r2v = jnp.where(c1, r1v, nv2)
            r2i = jnp.where(c1, r1i, ni2)
            r1v = nv1
            r1i = ni1

        ovals_ref[:, 0:1] = r1v
        ovals_ref[:, 1:2] = r2v
        ovals_ref[:, 2:3] = r3v
        oidx_ref[:, 0:1] = r1i
        oidx_ref[:, 1:2] = r2i
        oidx_ref[:, 2:3] = r3i


_NC = 2
_NS = 16
_NW = _NC * _NS
_LPW = (B * NUM_LM) // _NW
_EPW = _LPW * 3
_GPW = _LPW // 16


def _sc_combine(idx_hbm, vals_hbm, pdis_hbm, plm_hbm, tlm_hbm,
                out_hbm, loss_hbm,
                idx_v, row_v, vals_v, plm_v, tlm_v, rows_v, out_v, acc_v, sem):
    wid = lax.axis_index("s") * _NC + lax.axis_index("c")
    lbase = wid * _LPW

    pltpu.sync_copy(plm_hbm.at[pl.ds(lbase, _LPW), :], plm_v)
    pltpu.sync_copy(tlm_hbm.at[pl.ds(lbase, _LPW), :], tlm_v)
    pltpu.sync_copy(vals_hbm.at[pl.ds(lbase, _LPW), :], vals_v)
    pltpu.sync_copy(idx_hbm.at[pl.ds(lbase, _LPW), :], idx_v)

    lane = lax.iota(jnp.int32, 16)
    for i in range(_EPW // 16):
        pos = i * 16 + lane
        row_v[i // 6, pl.ds((i % 6) * 16, 16)] = (
            plsc.load_gather(idx_v, [pos // 3, pos % 3]) >> 5)

    pltpu.async_copy(pdis_hbm.at[row_v.at[0]], rows_v.at[pl.ds(0, 96)], sem)
    pltpu.async_copy(pdis_hbm.at[row_v.at[1]], rows_v.at[pl.ds(96, 96)], sem)
    pltpu.make_async_copy(pdis_hbm.at[row_v.at[0]],
                          rows_v.at[pl.ds(0, 96)], sem).wait()
    pltpu.make_async_copy(pdis_hbm.at[row_v.at[1]],
                          rows_v.at[pl.ds(96, 96)], sem).wait()

    acc = jnp.zeros((16,), jnp.float32)
    ccs = [jnp.full((16,), c, jnp.int32) for c in range(3)]
    for g in range(_GPW):
        lidx = g * 16 + lane
        e0 = 3 * lidx
        v = [plsc.load_gather(vals_v, [lidx, ccs[t]]) for t in range(3)]
        k = [1.0 / (vt + 1e-8) for vt in v]
        s = k[0] + k[1] + k[2]
        w = [kt / s for kt in k]
        col = [(plsc.load_gather(idx_v, [lidx, ccs[t]]) & 31) << 2
               for t in range(3)]
        for c in range(3):
            p_c = (w[0] * plsc.load_gather(rows_v, [e0 + 0, col[0] + c])
                   + w[1] * plsc.load_gather(rows_v, [e0 + 1, col[1] + c])
                   + w[2] * plsc.load_gather(rows_v, [e0 + 2, col[2] + c]))
            pred_c = plsc.load_gather(plm_v, [lidx, ccs[c]]) + p_c
            plsc.store_scatter(out_v, [lidx, ccs[c]], pred_c)
            err = plsc.load_gather(tlm_v, [lidx, ccs[c]]) - pred_c
            acc = acc + err * err

    acc_v[...] = acc
    pltpu.sync_copy(out_v, out_hbm.at[pl.ds(lbase, _LPW), :])
    pltpu.sync_copy(acc_v, loss_hbm.at[pl.ds(wid * 16, 16)])


_PACK_BLK = 512


def _sc_pack(pdis_hbm, out_hbm, in_v, out_v):
    wid = lax.axis_index("s") * _NC + lax.axis_index("c")
    ppw = N // _NW
    lane = lax.iota(jnp.int32, 16)
    lp = lane >> 2
    lc = jnp.minimum(lane & 3, 2)
    for blk in range(ppw // _PACK_BLK):
        src = wid * ppw + blk * _PACK_BLK
        dst = wid * (ppw // 32) + blk * (_PACK_BLK // 32)
        pltpu.sync_copy(pdis_hbm.at[pl.ds(src, _PACK_BLK), :], in_v)

        def body(r, _):
            p0 = r * 32
            for k in range(8):
                out_v[r, pl.ds(k * 16, 16)] = plsc.load_gather(
                    in_v, [p0 + 4 * k + lp, lc])
            return 0

        lax.fori_loop(0, _PACK_BLK // 32, body, 0)
        pltpu.sync_copy(
            out_v, out_hbm.at[pl.ds(dst, _PACK_BLK // 32), :])


_SC_CALL = None
_SC_PACK_CALL = None


def _get_sc_pack():
    global _SC_PACK_CALL
    if _SC_PACK_CALL is None:
        _SC_PACK_CALL = functools.partial(
            pl.kernel,
            mesh=plsc.VectorSubcoreMesh(core_axis_name="c",
                                        subcore_axis_name="s",
                                        num_cores=_NC, num_subcores=_NS),
            out_type=jax.ShapeDtypeStruct((N // 32, 128), jnp.float32),
            scratch_types=[
                pltpu.VMEM((_PACK_BLK, 3), jnp.float32),
                pltpu.VMEM((_PACK_BLK // 32, 128), jnp.float32),
            ],
            compiler_params=pltpu.CompilerParams(needs_layout_passes=False),
        )(_sc_pack)
    return _SC_PACK_CALL


def _get_sc_combine():
    global _SC_CALL
    if _SC_CALL is None:
        _SC_CALL = functools.partial(
            pl.kernel,
            mesh=plsc.VectorSubcoreMesh(core_axis_name="c",
                                        subcore_axis_name="s",
                                        num_cores=_NC, num_subcores=_NS),
            out_type=[
                jax.ShapeDtypeStruct((B * NUM_LM, 3), jnp.float32),
                jax.ShapeDtypeStruct((_NW * 16,), jnp.float32),
            ],
            scratch_types=[
                pltpu.VMEM((_LPW, 3), jnp.int32),
                pltpu.VMEM((2, 96), jnp.int32),
                pltpu.VMEM((_LPW, 3), jnp.float32),
                pltpu.VMEM((_LPW, 3), jnp.float32),
                pltpu.VMEM((_LPW, 3), jnp.float32),
                pltpu.VMEM((_EPW, 128), jnp.float32),
                pltpu.VMEM((_LPW, 3), jnp.float32),
                pltpu.VMEM((16,), jnp.float32),
                pltpu.SemaphoreType.DMA,
            ],
            compiler_params=pltpu.CompilerParams(needs_layout_passes=False),
        )(_sc_combine)
    return _SC_CALL


def _loss_reduce_kernel(parts_ref, loss_ref):
    loss_ref[0, 0] = jnp.sum(parts_ref[...]) / (B * NUM_LM)


def kernel(pred_dis, pre_xyz, pre_lm, target_lm, batch):
    batch = batch.astype(jnp.int32)
    pts_t = pre_xyz.T

    bids = jnp.arange(B, dtype=jnp.int32)
    starts = jnp.searchsorted(batch, bids, side="left").astype(jnp.int32)
    ends = jnp.searchsorted(batch, bids, side="right").astype(jnp.int32)
    start_c = starts // C
    end_c = jnp.where(ends > starts, (ends + C - 1) // C, start_c)
    counts = end_c - start_c
    cum = jnp.cumsum(counts)
    off = jnp.concatenate([jnp.zeros((1,), jnp.int32), cum[:-1]])
    vs = jnp.arange(V, dtype=jnp.int32)
    vb = jnp.searchsorted(cum, vs, side="right").astype(jnp.int32)
    vbc = jnp.minimum(vb, B - 1)
    vc = jnp.clip(start_c[vbc] + vs - off[vbc], 0, NCH - 1)
    scalars = jnp.concatenate([vb, vc, starts, ends])

    grid_spec = pltpu.PrefetchScalarGridSpec(
        num_scalar_prefetch=1,
        grid=(V,),
        in_specs=[
            pl.BlockSpec((NUM_LM, 3), lambda v, s: (jnp.minimum(s[v], B - 1), 0)),
            pl.BlockSpec((3, C), lambda v, s: (0, s[V + v])),
        ],
        out_specs=[
            pl.BlockSpec((NUM_LM, 3), lambda v, s: (jnp.minimum(s[v], B - 1), 0)),
            pl.BlockSpec((NUM_LM, 3), lambda v, s: (jnp.minimum(s[v], B - 1), 0)),
        ],
    )
    top_vals, top_idx = pl.pallas_call(
        _topk_kernel,
        grid_spec=grid_spec,
        out_shape=[
            jax.ShapeDtypeStruct((B * NUM_LM, 3), jnp.float32),
            jax.ShapeDtypeStruct((B * NUM_LM, 3), jnp.int32),
        ],
    )(scalars, pre_lm, pts_t)

    pdis_pack = _get_sc_pack()(pred_dis)
    pred_lm, loss_parts = _get_sc_combine()(
        top_idx,
        top_vals,
        pdis_pack,
        pre_lm,
        target_lm,
    )

    loss = pl.pallas_call(
        _loss_reduce_kernel,
        in_specs=[pl.BlockSpec(memory_space=pltpu.VMEM)],
        out_specs=pl.BlockSpec(memory_space=pltpu.SMEM),
        out_shape=jax.ShapeDtypeStruct((1, 1), jnp.float32),
    )(loss_parts.reshape(_NW, 16))

    return loss[0, 0], pred_lm
